# baseline (speedup 1.0000x reference)
import functools
import math

import numpy as np
import jax
import jax.numpy as jnp
from jax import lax
from jax.experimental import pallas as pl
from jax.experimental.pallas import tpu as pltpu
from jax.experimental.pallas import tpu_sc as plsc

VOCAB = 259
D = 128
B = 4096
S = 200
VP = 264
NC, NS = 2, 16
NW = NC * NS
TOK = B * S
TPW = TOK // NW
CH = 128
NCH = TPW // CH
NP = NCH // 2
NRING = 3

LANES = 128
ROWS = TOK // LANES
XBLK = 1280

SBLK = 40


def _pe_np():
    position = np.arange(0, S, dtype=np.float32)[:, None]
    div_term = np.exp(
        np.arange(0, D, 2, dtype=np.float32) * (-math.log(10000.0) / D))
    pe = np.zeros((S, D), dtype=np.float32)
    pe[:, 0::2] = np.sin(position * div_term)
    pe[:, 1::2] = np.cos(position * div_term)
    return pe


_PE = _pe_np()


def _prep_body(tab_ref, pe_ref, g_ref, b_ref, x_ref, f_ref, idx_ref):
    h = tab_ref[...][None, :, :] + pe_ref[...][:, None, :]
    m = jnp.mean(h, axis=-1, keepdims=True)
    r = h - m
    v = jnp.mean(r * r, axis=-1, keepdims=True)
    f_ref[...] = r * lax.rsqrt(v + 1e-5) * g_ref[...] + b_ref[...]
    pid = pl.program_id(0)
    rr = lax.broadcasted_iota(jnp.int32, (XBLK, LANES), 0)
    cc = lax.broadcasted_iota(jnp.int32, (XBLK, LANES), 1)
    t = (pid * XBLK + rr) * LANES + cc
    idx_ref[...] = x_ref[...] + (t % S) * VP


def _prep_kernel(table_pad, pe, gamma2, beta2, xflat):
    return pl.pallas_call(
        _prep_body,
        grid=(S // SBLK,),
        in_specs=[
            pl.BlockSpec((VP, D), lambda i: (0, 0)),
            pl.BlockSpec((SBLK, D), lambda i: (i, 0)),
            pl.BlockSpec((1, D), lambda i: (0, 0)),
            pl.BlockSpec((1, D), lambda i: (0, 0)),
            pl.BlockSpec((XBLK, LANES), lambda i: (i, 0)),
        ],
        out_specs=[
            pl.BlockSpec((SBLK, VP, D), lambda i: (i, 0, 0)),
            pl.BlockSpec((XBLK, LANES), lambda i: (i, 0)),
        ],
        out_shape=[
            jax.ShapeDtypeStruct((S, VP, D), jnp.float32),
            jax.ShapeDtypeStruct((ROWS, LANES), jnp.int32),
        ],
    )(table_pad, pe, gamma2, beta2, xflat)


def _sc_gather(f_flat, idx3):
    mesh = plsc.VectorSubcoreMesh(core_axis_name="c", subcore_axis_name="s")

    @functools.partial(
        pl.kernel,
        out_type=jax.ShapeDtypeStruct((TOK, D), jnp.float32),
        mesh=mesh,
        scratch_types=[
            pltpu.VMEM((NCH, CH), jnp.int32),
            pltpu.VMEM((NRING, 2 * CH, D), jnp.float32),
            pltpu.SemaphoreType.DMA,
            pltpu.SemaphoreType.DMA,
            pltpu.SemaphoreType.DMA,
            pltpu.SemaphoreType.DMA,
            pltpu.SemaphoreType.DMA,
            pltpu.SemaphoreType.DMA,
        ],
    )
    def k(f_hbm, idx_hbm, out_hbm, idx_v, rows_v, g0, g1, g2, w0, w1, w2):
        gsems = [g0, g1, g2]
        wsems = [w0, w1, w2]
        wid = lax.axis_index("s") * NC + lax.axis_index("c")
        base = wid * TPW

        pltpu.sync_copy(idx_hbm.at[wid], idx_v)

        def gpair(j2, slot):
            for h in range(2):
                pltpu.async_copy(f_hbm.at[idx_v.at[2 * j2 + h]],
                                 rows_v.at[slot, pl.ds(h * CH, CH)],
                                 gsems[slot])

        def wwait(slot):
            pltpu.make_async_copy(
                rows_v.at[slot],
                out_hbm.at[pl.ds(pl.multiple_of(base, CH), 2 * CH)],
                wsems[slot]).wait()

        def fetch(j2, slot):
            wwait(slot)
            gpair(j2, slot)

        def drain(j2, slot):
            for h in range(2):
                pltpu.make_async_copy(f_hbm.at[idx_v.at[2 * j2 + h]],
                                      rows_v.at[slot, pl.ds(h * CH, CH)],
                                      gsems[slot]).wait()
            off = pl.multiple_of(base + j2 * 2 * CH, CH)
            pltpu.async_copy(rows_v.at[slot],
                             out_hbm.at[pl.ds(off, 2 * CH)], wsems[slot])

        gpair(0, 0)
        gpair(1, 1)
        drain(0, 0)
        gpair(2, 2)

        def body(i, carry):
            k0 = i * NRING + 1
            for p in range(NRING):
                k = k0 + p
                drain(k, (1 + p) % NRING)
                fetch(k + 2, p)
            return carry

        lax.fori_loop(0, (NP - 4) // NRING, body, 0)

        drain(NP - 3, (NP - 3) % NRING)
        drain(NP - 2, (NP - 2) % NRING)
        fetch(NP - 1, (NP - 1) % NRING)
        drain(NP - 1, (NP - 1) % NRING)
        for p in range(NRING):
            wwait(p)

    return k(f_flat, idx3)


def kernel(x, table, gamma, beta):
    x = x.astype(jnp.int32)
    table_pad = jnp.zeros((VP, D), table.dtype).at[:VOCAB].set(table)
    pe = jnp.asarray(_PE)
    f, idx = _prep_kernel(table_pad, pe, gamma.reshape(1, D),
                          beta.reshape(1, D), x.reshape(ROWS, LANES))
    f_flat = f.reshape(S * VP, D)
    out = _sc_gather(f_flat, idx.reshape(NW, NCH, CH))
    return out.reshape(B, S, D)
